# baseline (speedup 1.0000x reference)
import jax
import jax.numpy as jnp
from jax.experimental import pallas as pl
from jax.experimental.pallas import tpu as pltpu


def _build_tc(B, N, BN, BPB):
    NBLK = N // BN
    GB = B // BPB

    def body(p_ref, g_ref, out_ref):
        gb = pl.program_id(0)
        for j in range(BPB):
            pblk = p_ref[j]
            gblk = jnp.transpose(-2.0 * g_ref[j])
            pn = jnp.sum(pblk * pblk, axis=1, keepdims=True)
            gx, gy, gz = gblk[0:1, :], gblk[1:2, :], gblk[2:3, :]
            gn = 0.25 * (gx * gx + gy * gy + gz * gz)
            g16 = gblk.astype(jnp.bfloat16)
            colmin = jnp.full((1, N), jnp.inf, jnp.float32)
            xsum = jnp.float32(0.0)
            for i in range(NBLK):
                lo, hi = i * BN, (i + 1) * BN
                cprime = jnp.dot(pblk[lo:hi, :].astype(jnp.bfloat16), g16,
                                 preferred_element_type=jnp.float32)
                d2 = (pn[lo:hi, :] + gn) + cprime
                xs = jnp.min(d2, axis=1, keepdims=True)
                xsum = xsum + jnp.sum(jnp.maximum(xs, 0.0))
                colmin = jnp.minimum(colmin,
                                     jnp.min(d2, axis=0, keepdims=True))
            out_ref[gb * BPB + j, 0] = xsum
            out_ref[gb * BPB + j, 1] = jnp.sum(jnp.maximum(colmin, 0.0))

    return pl.pallas_call(
        body,
        grid=(GB,),
        in_specs=[
            pl.BlockSpec((BPB, N, 3), lambda g: (g, 0, 0)),
            pl.BlockSpec((BPB, N, 3), lambda g: (g, 0, 0)),
        ],
        out_specs=pl.BlockSpec((B, 2), lambda g: (0, 0),
                               memory_space=pltpu.SMEM),
        out_shape=jax.ShapeDtypeStruct((B, 2), jnp.float32),
    )


_chamfer_tc = _build_tc(8, 2048, 1024, 2)


def kernel(pred_points, gt_points):
    B, N, _ = pred_points.shape
    pred = pred_points.astype(jnp.float32)
    gt = gt_points.astype(jnp.float32)
    parts = _chamfer_tc(pred, gt)
    return jnp.sum(parts) / (B * N)


# speedup vs baseline: 1.1714x; 1.1714x over previous
import jax
import jax.numpy as jnp
from jax.experimental import pallas as pl
from jax.experimental.pallas import tpu as pltpu


def _build_tc(B, N, BN, BPB):
    NBLK = N // BN
    GB = B // BPB

    def body(p_ref, g_ref, out_ref):
        gb = pl.program_id(0)
        for j in range(BPB):
            pblk = p_ref[j]
            gblk = g_ref[j]
            pn = jnp.sum(pblk * pblk, axis=1, keepdims=True)
            gx, gy, gz = gblk[0:1, :], gblk[1:2, :], gblk[2:3, :]
            gn = 0.25 * (gx * gx + gy * gy + gz * gz)
            g16 = gblk.astype(jnp.bfloat16)
            colmin = jnp.full((1, N), jnp.inf, jnp.float32)
            xsum = jnp.float32(0.0)
            for i in range(NBLK):
                lo, hi = i * BN, (i + 1) * BN
                cprime = jnp.dot(pblk[lo:hi, :].astype(jnp.bfloat16), g16,
                                 preferred_element_type=jnp.float32)
                d2 = (pn[lo:hi, :] + gn) + cprime
                xs = jnp.min(d2, axis=1, keepdims=True)
                xsum = xsum + jnp.sum(jnp.maximum(xs, 0.0))
                colmin = jnp.minimum(colmin,
                                     jnp.min(d2, axis=0, keepdims=True))
            out_ref[gb * BPB + j, 0] = xsum
            out_ref[gb * BPB + j, 1] = jnp.sum(jnp.maximum(colmin, 0.0))

    return pl.pallas_call(
        body,
        grid=(GB,),
        in_specs=[
            pl.BlockSpec((BPB, N, 3), lambda g: (g, 0, 0)),
            pl.BlockSpec((BPB, 3, N), lambda g: (g, 0, 0)),
        ],
        out_specs=pl.BlockSpec((B, 2), lambda g: (0, 0),
                               memory_space=pltpu.SMEM),
        out_shape=jax.ShapeDtypeStruct((B, 2), jnp.float32),
    )


_chamfer_tc = _build_tc(8, 2048, 1024, 2)


def kernel(pred_points, gt_points):
    B, N, _ = pred_points.shape
    pred = pred_points.astype(jnp.float32)
    gt = gt_points.astype(jnp.float32)
    g2T = jnp.swapaxes(-2.0 * gt, 1, 2)
    parts = _chamfer_tc(pred, g2T)
    return jnp.sum(parts) / (B * N)
